# baseline (speedup 1.0000x reference)
import functools

import jax
import jax.numpy as jnp
from jax import lax
from jax.experimental import pallas as pl
from jax.experimental.pallas import tpu as pltpu
from jax.experimental.pallas import tpu_sc as plsc

N = 100000
NP = 100096
CH = 8
E = 1600000
NC = 2
NS = 16
NW = NC * NS
CHUNK = 128
NCHUNKS = E // CHUNK
CPW = NCHUNKS // NW
EXTRA = NCHUNKS - CPW * NW
K = 39
SUPER = CPW // K
RPT = NP // NS
DR = NP // NW
B0, B1A, B1B = 1568, 1560, 1464

_mesh = plsc.VectorSubcoreMesh(core_axis_name="c", subcore_axis_name="s",
                               num_cores=NC, num_subcores=NS)
_agg_params = pltpu.CompilerParams(use_tc_tiling_on_sc=False)
_cmp_params = pltpu.CompilerParams(use_tc_tiling_on_sc=False,
                                   needs_layout_passes=False)
_P = jax.ShapeDtypeStruct((NP, CH), jnp.float32)


def _newton_rsqrt(v):
    bi = plsc.bitcast(v, jnp.int32)
    y = plsc.bitcast(jnp.int32(0x5F3759DF) - lax.shift_right_logical(bi, 1),
                     jnp.float32)
    vh = 0.5 * v
    for _ in range(3):
        y = y * (1.5 - vh * y * y)
    return y


def _flatpos(i):
    iota = lax.iota(jnp.int32, 16)
    return lax.shift_right_logical(iota, 3) + 2 * i, iota & 7


@functools.partial(
    pl.kernel,
    out_type=[_P, _P],
    mesh=_mesh,
    scratch_types=[
        pltpu.VMEM((K, CHUNK), jnp.int32),
        pltpu.VMEM((CHUNK, CH), jnp.float32),
        pltpu.VMEM_SHARED((NP, CH), jnp.float32),
        pltpu.SemaphoreType.DMA,
    ],
    compiler_params=_agg_params,
)
def _sc_deg(ei3, zeros_hbm, ones_hbm, out0, out1, didx, ones_v, acc_sh, ssem):
    c = lax.axis_index("c")
    s = lax.axis_index("s")
    wid = s * NC + c
    pltpu.sync_copy(zeros_hbm, acc_sh.at[pl.ds(s * RPT, RPT)])
    pltpu.sync_copy(ones_hbm, ones_v)
    plsc.subcore_barrier()

    @pl.when(wid < EXTRA)
    def _():
        row = NW * CPW + wid
        pltpu.sync_copy(ei3.at[1, row], didx.at[0])
        pltpu.async_copy(ones_v, acc_sh.at[didx.at[0]], ssem, add=True).wait()

    @pl.loop(0, SUPER)
    def _(g):
        row0 = wid * CPW + g * K
        pltpu.sync_copy(ei3.at[1, pl.ds(row0, K)], didx)
        descs = [
            pltpu.async_copy(ones_v, acc_sh.at[didx.at[j]], ssem, add=True)
            for j in range(K)
        ]
        for dsc in descs:
            dsc.wait()

    plsc.subcore_barrier()

    @pl.when(c == 0)
    def _():
        pltpu.sync_copy(acc_sh.at[pl.ds(s * RPT, RPT)],
                        out0.at[pl.ds(s * RPT, RPT)])

    @pl.when(c == 1)
    def _():
        pltpu.sync_copy(acc_sh.at[pl.ds(s * RPT, RPT)],
                        out1.at[pl.ds(s * RPT, RPT)])


@functools.partial(
    pl.kernel,
    out_type=[_P, _P],
    mesh=_mesh,
    scratch_types=[
        pltpu.VMEM((K, CHUNK), jnp.int32),
        pltpu.VMEM((K, CHUNK), jnp.int32),
        pltpu.VMEM((K, CHUNK, CH), jnp.float32),
        pltpu.VMEM_SHARED((NP, CH), jnp.float32),
        pltpu.SemaphoreType.DMA,
        pltpu.SemaphoreType.DMA,
    ],
    compiler_params=_agg_params,
)
def _sc_agg(ei3, u_hbm, zeros_hbm, outa, outb,
            sidx, didx, rows, acc_sh, gsem, ssem):
    c = lax.axis_index("c")
    s = lax.axis_index("s")
    wid = s * NC + c
    pltpu.sync_copy(zeros_hbm, acc_sh.at[pl.ds(s * RPT, RPT)])
    plsc.subcore_barrier()

    @pl.when(wid < EXTRA)
    def _():
        row = NW * CPW + wid
        pltpu.sync_copy(ei3.at[0, row], sidx.at[0])
        pltpu.sync_copy(ei3.at[1, row], didx.at[0])
        pltpu.async_copy(u_hbm.at[sidx.at[0]], rows.at[0], gsem).wait()
        pltpu.async_copy(rows.at[0], acc_sh.at[didx.at[0]], ssem,
                         add=True).wait()

    @pl.loop(0, SUPER)
    def _(g):
        row0 = wid * CPW + g * K
        pltpu.sync_copy(ei3.at[0, pl.ds(row0, K)], sidx)
        pltpu.sync_copy(ei3.at[1, pl.ds(row0, K)], didx)
        gds = [pltpu.async_copy(u_hbm.at[sidx.at[j]], rows.at[j], gsem)
               for j in range(K)]
        sds = []
        for j in range(K):
            gds[j].wait()
            sds.append(
                pltpu.async_copy(rows.at[j], acc_sh.at[didx.at[j]], ssem,
                                 add=True))
        for dsc in sds:
            dsc.wait()

    plsc.subcore_barrier()

    @pl.when(c == 0)
    def _():
        pltpu.sync_copy(acc_sh.at[pl.ds(s * RPT, RPT)],
                        outa.at[pl.ds(s * RPT, RPT)])

    @pl.when(c == 1)
    def _():
        pltpu.sync_copy(acc_sh.at[pl.ds(s * RPT, RPT)],
                        outb.at[pl.ds(s * RPT, RPT)])


@functools.partial(
    pl.kernel,
    out_type=[_P, _P],
    mesh=_mesh,
    scratch_types=[
        pltpu.VMEM((B0, CH), jnp.float32),
        pltpu.VMEM((B0, CH), jnp.float32),
        pltpu.VMEM((B0, 3), jnp.float32),
        pltpu.VMEM((B0, CH), jnp.float32),
        pltpu.VMEM((B0, CH), jnp.float32),
    ],
    compiler_params=_cmp_params,
)
def _sc_prep(dp0, dp1, x_hbm, u1t, d8t, pa, pb, px, pu, pd):
    c = lax.axis_index("c")
    s = lax.axis_index("s")
    wid = s * NC + c

    def block(off, size, xsize):
        base = wid * DR + off
        pltpu.sync_copy(dp0.at[pl.ds(base, size)], pa.at[pl.ds(0, size)])
        pltpu.sync_copy(dp1.at[pl.ds(base, size)], pb.at[pl.ds(0, size)])
        pltpu.sync_copy(x_hbm.at[pl.ds(base, xsize)], px.at[pl.ds(0, xsize)])

        @pl.loop(0, size // 2)
        def _(i):
            rowv, colv = _flatpos(i)
            a = plsc.load_gather(pa, [rowv, colv])
            b = plsc.load_gather(pb, [rowv, colv])
            dv = _newton_rsqrt(a + b + 1.0)
            xg = plsc.load_gather(px, [rowv, jnp.minimum(colv, 2)])
            plsc.store_scatter(pd, [rowv, colv], dv)
            plsc.store_scatter(pu, [rowv, colv], dv * xg)

        pltpu.sync_copy(pu.at[pl.ds(0, size)], u1t.at[pl.ds(base, size)])
        pltpu.sync_copy(pd.at[pl.ds(0, size)], d8t.at[pl.ds(base, size)])

    block(0, B0, B0)

    @pl.when(wid == NW - 1)
    def _():
        block(B0, B1A, B1B)

    @pl.when(wid < NW - 1)
    def _():
        block(B0, B1A, B1A)


DS = 1564


@functools.partial(
    pl.kernel,
    out_type=_P,
    mesh=_mesh,
    scratch_types=[
        pltpu.VMEM((DS, CH), jnp.float32),
        pltpu.VMEM((DS, CH), jnp.float32),
        pltpu.VMEM((DS, CH), jnp.float32),
        pltpu.VMEM((DS, CH), jnp.float32),
        pltpu.VMEM((DS, CH), jnp.float32),
        pltpu.VMEM((DS * CH,), jnp.float32),
        pltpu.VMEM((DS * CH,), jnp.float32),
        pltpu.VMEM((64,), jnp.float32),
        pltpu.VMEM((CH, 32), jnp.float32),
        pltpu.VMEM((32,), jnp.float32),
        pltpu.VMEM((3, 32), jnp.float32),
        pltpu.SMEM((CH, 32), jnp.float32),
        pltpu.SMEM((32,), jnp.float32),
        pltpu.SMEM((3, 32), jnp.float32),
    ],
    compiler_params=_cmp_params,
)
def _sc_dense(d8t, s1a, s1b, u1t, w1_hbm, b1_hbm, w2t_hbm, u2t,
              qd, qs, qt, qu, qo, ybuf, dbuf, zbuf, w1v, b1vv, w2v,
              w1, b1v, w2):
    c = lax.axis_index("c")
    s = lax.axis_index("s")
    wid = s * NC + c
    pltpu.sync_copy(w1_hbm, w1v)
    pltpu.sync_copy(b1_hbm, b1vv)
    pltpu.sync_copy(w2t_hbm, w2v)
    for r in range(CH):
        for cb in (0, 16):
            vv = w1v[r, pl.ds(cb, 16)]
            for j in range(16):
                w1[r, cb + j] = vv[j]
    for cb in (0, 16):
        vv = b1vv[pl.ds(cb, 16)]
        for j in range(16):
            b1v[cb + j] = vv[j]
    for r in range(3):
        for cb in (0, 16):
            vv = w2v[r, pl.ds(cb, 16)]
            for j in range(16):
                w2[r, cb + j] = vv[j]
    iota = lax.iota(jnp.int32, 16)
    zero16 = jnp.zeros((16,), jnp.float32)
    for t in range(4):
        zbuf[pl.ds(t * 16, 16)] = zero16
    colv8 = iota & 7
    rbase = lax.shift_right_logical(iota, 3)
    basepat = jnp.where(colv8 < 3, colv8 * 16 + rbase, 48)

    for off in (0, DS):
        base = wid * DR + off
        pltpu.sync_copy(d8t.at[pl.ds(base, DS)], qd)
        pltpu.sync_copy(s1a.at[pl.ds(base, DS)], qs)
        pltpu.sync_copy(s1b.at[pl.ds(base, DS)], qt)
        pltpu.sync_copy(u1t.at[pl.ds(base, DS)], qu)

        @pl.loop(0, DS // 2)
        def _(i):
            rowv, colv = _flatpos(i)
            dv = plsc.load_gather(qd, [rowv, colv])
            sv = (plsc.load_gather(qs, [rowv, colv])
                  + plsc.load_gather(qt, [rowv, colv])
                  + plsc.load_gather(qu, [rowv, colv]))
            dbuf[pl.ds(i * 16, 16)] = dv
            ybuf[pl.ds(i * 16, 16)] = dv * sv

        NGROUPS = DS // 32 + 1

        @pl.loop(0, NGROUPS)
        def _(g):
            ng = jnp.minimum(g * 32, DS - 32)
            fb = ng * CH
            yca = [plsc.load_gather(ybuf, [iota * CH + (fb + cc)])
                   for cc in range(CH)]
            ycb = [plsc.load_gather(ybuf, [iota * CH + (fb + 128 + cc)])
                   for cc in range(CH)]
            za = [zero16, zero16, zero16]
            zb = [zero16, zero16, zero16]
            for k in range(32):
                wk = [w1[cc, k] for cc in range(CH)]
                ha = yca[0] * wk[0]
                hb = ycb[0] * wk[0]
                for cc in range(1, CH):
                    ha = ha + yca[cc] * wk[cc]
                    hb = hb + ycb[cc] * wk[cc]
                bk = b1v[k]
                ha = jnp.maximum(ha + bk, 0.0)
                hb = jnp.maximum(hb + bk, 0.0)
                for r in range(3):
                    wr = w2[r, k]
                    za[r] = za[r] + ha * wr
                    zb[r] = zb[r] + hb * wr
            for half, zz in ((0, za), (1, zb)):
                zbuf[pl.ds(0, 16)] = zz[0]
                zbuf[pl.ds(16, 16)] = zz[1]
                zbuf[pl.ds(32, 16)] = zz[2]
                for v in range(CH):
                    zg = plsc.load_gather(zbuf, [basepat + 2 * v])
                    dv = dbuf[pl.ds(fb + half * 128 + v * 16, 16)]
                    rowv = rbase + (ng + half * 16 + 2 * v)
                    plsc.store_scatter(qo, [rowv, colv8], dv * zg)

        pltpu.sync_copy(qo, u2t.at[pl.ds(base, DS)])


@functools.partial(
    pl.kernel,
    out_type=jax.ShapeDtypeStruct((N, 3), jnp.float32),
    mesh=_mesh,
    scratch_types=[
        pltpu.VMEM((B0, CH), jnp.float32),
        pltpu.VMEM((B0, CH), jnp.float32),
        pltpu.VMEM((B0, CH), jnp.float32),
        pltpu.VMEM((B0, CH), jnp.float32),
        pltpu.VMEM((B0 * CH,), jnp.float32),
        pltpu.VMEM((B0, 3), jnp.float32),
        pltpu.VMEM((16,), jnp.float32),
    ],
    compiler_params=_cmp_params,
)
def _sc_final(d8t, s2a, s2b, u2t, b2_hbm, dxout,
              rd, rs, rt, ru, vbuf, dxbuf, b2b):
    c = lax.axis_index("c")
    s = lax.axis_index("s")
    wid = s * NC + c
    pltpu.sync_copy(b2_hbm, b2b)
    iota = lax.iota(jnp.int32, 16)
    colv8 = iota & 7
    b2vec = plsc.load_gather(b2b, [jnp.where(colv8 < 3, colv8, 3)])
    pats, dxrow, dxcol = [], [], []
    for r in range(3):
        f = iota + 16 * r
        n3 = f // 3
        c3 = f - 3 * n3
        pats.append(CH * n3 + c3)
        dxrow.append(n3)
        dxcol.append(c3)

    def block(off, size):
        base = wid * DR + off
        pltpu.sync_copy(d8t.at[pl.ds(base, size)], rd.at[pl.ds(0, size)])
        pltpu.sync_copy(s2a.at[pl.ds(base, size)], rs.at[pl.ds(0, size)])
        pltpu.sync_copy(s2b.at[pl.ds(base, size)], rt.at[pl.ds(0, size)])
        pltpu.sync_copy(u2t.at[pl.ds(base, size)], ru.at[pl.ds(0, size)])

        @pl.loop(0, size // 2)
        def _(i):
            rowv, colv = _flatpos(i)
            dv = plsc.load_gather(rd, [rowv, colv])
            sv = (plsc.load_gather(rs, [rowv, colv])
                  + plsc.load_gather(rt, [rowv, colv])
                  + plsc.load_gather(ru, [rowv, colv]))
            vbuf[pl.ds(i * 16, 16)] = dv * sv + b2vec

        @pl.loop(0, size // 16 + 1)
        def _(q):
            mq = jnp.minimum(q * 16, size - 16)
            for r in range(3):
                dxv = plsc.load_gather(vbuf, [pats[r] + CH * mq])
                plsc.store_scatter(dxbuf, [dxrow[r] + mq, dxcol[r]], dxv)

        pltpu.sync_copy(dxbuf.at[pl.ds(0, size)],
                        dxout.at[pl.ds(base, size)])

    block(0, B0)

    @pl.when(wid == NW - 1)
    def _():
        block(B0, B1B)

    @pl.when(wid < NW - 1)
    def _():
        block(B0, B1A)


def kernel(x, edge_index, W1, b1, W2, b2):
    ei3 = edge_index.astype(jnp.int32).reshape(2, NCHUNKS, CHUNK)
    W1p = jnp.pad(W1, ((0, CH - 3), (0, 0)))
    W2T = W2.T
    b2p = jnp.pad(b2, (0, 13))
    zeros8 = jnp.zeros((RPT, CH), jnp.float32)
    ones8 = jnp.ones((CHUNK, CH), jnp.float32)

    dp0, dp1 = _sc_deg(ei3, zeros8, ones8)
    u1t, d8t = _sc_prep(dp0, dp1, x)
    s1a, s1b = _sc_agg(ei3, u1t, zeros8)
    u2t = _sc_dense(d8t, s1a, s1b, u1t, W1p, b1, W2T)
    s2a, s2b = _sc_agg(ei3, u2t, zeros8)
    return _sc_final(d8t, s2a, s2b, u2t, b2p)
